# baseline (speedup 1.0000x reference)
import functools

import jax
import jax.numpy as jnp
from jax import lax
from jax.experimental import pallas as pl
from jax.experimental.pallas import tpu as pltpu
from jax.experimental.pallas import tpu_sc as plsc

NUM_FIELDS = 26
CARD1 = 100001
D = 32
BATCH = 16384
NC, NS, L = 2, 16, 16
NW = NC * NS
BPW = BATCH // NW
IDX_MINOR = 128
G_PER_FIELD = BPW // IDX_MINOR

_mesh = plsc.VectorSubcoreMesh(
    core_axis_name="c", subcore_axis_name="s", num_cores=NC, num_subcores=NS
)


@functools.partial(
    pl.kernel,
    out_type=jax.ShapeDtypeStruct((BATCH, NUM_FIELDS, D), jnp.float32),
    mesh=_mesh,
    scratch_types=[
        pltpu.VMEM((BPW, NUM_FIELDS), jnp.int32),
        pltpu.VMEM((G_PER_FIELD, IDX_MINOR), jnp.int32),
        pltpu.VMEM((BPW, D), jnp.float32),
        pltpu.SemaphoreType.DMA,
    ],
    compiler_params=pltpu.CompilerParams(
        use_tc_tiling_on_sc=False, needs_layout_passes=False
    ),
)
def _emb_gather(x_hbm, tables_hbm, out_hbm, slab_v, idx_v, buf_v, sem):
    wid = lax.axis_index("s") * NC + lax.axis_index("c")
    b0 = wid * BPW
    pltpu.sync_copy(x_hbm.at[pl.ds(b0, BPW)], slab_v)

    def field_body(f, carry):
        fcol = jnp.full((L,), f, dtype=jnp.int32)
        for k in range(G_PER_FIELD):
            for m in range(IDX_MINOR // L):
                rows = lax.iota(jnp.int32, L) + (k * IDX_MINOR + m * L)
                idx_v[k, pl.ds(m * L, L)] = plsc.load_gather(
                    slab_v, [rows, fcol]
                )
        g_descs = [
            pltpu.async_copy(
                tables_hbm.at[f].at[idx_v.at[k]],
                buf_v.at[pl.ds(k * IDX_MINOR, IDX_MINOR)],
                sem,
            )
            for k in range(G_PER_FIELD)
        ]
        for d_ in g_descs:
            d_.wait()
        pltpu.sync_copy(buf_v, out_hbm.at[pl.ds(b0, BPW), f])
        return carry

    lax.fori_loop(0, NUM_FIELDS, field_body, 0)


def kernel(x, tables):
    return _emb_gather(x, tables)


# speedup vs baseline: 13.7221x; 13.7221x over previous
import functools

import jax
import jax.numpy as jnp
from jax import lax
from jax.experimental import pallas as pl
from jax.experimental.pallas import tpu as pltpu
from jax.experimental.pallas import tpu_sc as plsc

NUM_FIELDS = 26
CARD1 = 100001
D = 32
BATCH = 16384
NC, NS, L = 2, 16, 16
NW = NC * NS
HALF = BATCH // 2

_mesh = plsc.VectorSubcoreMesh(
    core_axis_name="c", subcore_axis_name="s", num_cores=NC, num_subcores=NS
)


@functools.partial(
    pl.kernel,
    out_type=jax.ShapeDtypeStruct((NUM_FIELDS, D, BATCH), jnp.float32),
    mesh=_mesh,
    scratch_types=[
        pltpu.VMEM((CARD1,), jnp.float32),
        pltpu.VMEM((1, HALF), jnp.int32),
        pltpu.VMEM((1, HALF), jnp.float32),
        pltpu.SemaphoreType.DMA,
    ],
    compiler_params=pltpu.CompilerParams(needs_layout_passes=False),
)
def _emb_gather(xt_hbm, tt_hbm, out_hbm, row_v, idx_v, buf_v, sem):
    d = lax.axis_index("s") * NC + lax.axis_index("c")

    def field_body(f, carry):
        pltpu.sync_copy(tt_hbm.at[f].at[d], row_v)

        def half_body(h, carry2):
            b0 = h * HALF
            pltpu.sync_copy(xt_hbm.at[pl.ds(f, 1), pl.ds(b0, HALF)], idx_v)

            def gather_body(m, carry3):
                idx16 = idx_v[0, pl.ds(m * L, L)]
                buf_v[0, pl.ds(m * L, L)] = plsc.load_gather(row_v, [idx16])
                return carry3

            lax.fori_loop(0, HALF // L, gather_body, 0)
            pltpu.sync_copy(
                buf_v, out_hbm.at[f].at[pl.ds(d, 1), pl.ds(b0, HALF)]
            )
            return carry2

        lax.fori_loop(0, 2, half_body, 0)
        return carry

    lax.fori_loop(0, NUM_FIELDS, field_body, 0)


def kernel(x, tables):
    xt = x.T
    tt = jnp.transpose(tables, (0, 2, 1))
    out_t = _emb_gather(xt, tt)
    return jnp.transpose(out_t, (2, 0, 1))


# speedup vs baseline: 20.0919x; 1.4642x over previous
import functools

import jax
import jax.numpy as jnp
from jax import lax
from jax.experimental import pallas as pl
from jax.experimental.pallas import tpu as pltpu
from jax.experimental.pallas import tpu_sc as plsc

NUM_FIELDS = 26
CARD1 = 100001
D = 32
BATCH = 16384
NC, NS, L = 2, 16, 16
NW = NC * NS
HALF = BATCH // 2

_mesh = plsc.VectorSubcoreMesh(
    core_axis_name="c", subcore_axis_name="s", num_cores=NC, num_subcores=NS
)


@functools.partial(
    pl.kernel,
    out_type=jax.ShapeDtypeStruct((NUM_FIELDS, D, BATCH), jnp.float32),
    mesh=_mesh,
    scratch_types=[
        pltpu.VMEM((CARD1,), jnp.float32),
        pltpu.VMEM((1, HALF), jnp.int32),
        pltpu.VMEM((1, HALF), jnp.float32),
        pltpu.SemaphoreType.DMA,
    ],
    compiler_params=pltpu.CompilerParams(needs_layout_passes=False),
)
def _emb_gather(xt_hbm, tt_hbm, out_hbm, row_v, idx_v, buf_v, sem):
    d = lax.axis_index("s") * NC + lax.axis_index("c")

    def field_body(f, carry):
        pltpu.sync_copy(tt_hbm.at[f].at[d], row_v)

        def half_body(h, carry2):
            b0 = h * HALF
            pltpu.sync_copy(xt_hbm.at[pl.ds(f, 1), pl.ds(b0, HALF)], idx_v)

            @plsc.parallel_loop(0, HALF // L, unroll=8)
            def gather_body(m):
                idx16 = idx_v[0, pl.ds(m * L, L)]
                buf_v[0, pl.ds(m * L, L)] = plsc.load_gather(row_v, [idx16])
            pltpu.sync_copy(
                buf_v, out_hbm.at[f].at[pl.ds(d, 1), pl.ds(b0, HALF)]
            )
            return carry2

        lax.fori_loop(0, 2, half_body, 0)
        return carry

    lax.fori_loop(0, NUM_FIELDS, field_body, 0)


def kernel(x, tables):
    xt = x.T
    tt = jnp.transpose(tables, (0, 2, 1))
    out_t = _emb_gather(xt, tt)
    return jnp.transpose(out_t, (2, 0, 1))


# speedup vs baseline: 23.1729x; 1.1533x over previous
import functools

import jax
import jax.numpy as jnp
from jax import lax
from jax.experimental import pallas as pl
from jax.experimental.pallas import tpu as pltpu
from jax.experimental.pallas import tpu_sc as plsc

NUM_FIELDS = 26
CARD1 = 100001
D = 32
BATCH = 16384
NC, NS, L = 2, 16, 16
NW = NC * NS
QB = 4096
NQ = BATCH // QB

_mesh = plsc.VectorSubcoreMesh(
    core_axis_name="c", subcore_axis_name="s", num_cores=NC, num_subcores=NS
)


@functools.partial(
    pl.kernel,
    out_type=jax.ShapeDtypeStruct((NUM_FIELDS, D, BATCH), jnp.float32),
    mesh=_mesh,
    scratch_types=[
        pltpu.VMEM((CARD1,), jnp.float32),
        pltpu.VMEM((1, BATCH), jnp.int32),
        pltpu.VMEM((2, QB), jnp.float32),
        pltpu.SemaphoreType.DMA,
        pltpu.SemaphoreType.DMA,
        pltpu.SemaphoreType.DMA,
        pltpu.SemaphoreType.DMA,
    ],
    compiler_params=pltpu.CompilerParams(needs_layout_passes=False),
)
def _emb_gather(xt_hbm, tt_hbm, out_hbm, row_v, idx_v, buf_v, sem_r, sem_i,
                sem_o0, sem_o1):
    d = lax.axis_index("s") * NC + lax.axis_index("c")
    f0 = lax.rem(d, NUM_FIELDS)
    pltpu.async_copy(tt_hbm.at[f0].at[d], row_v, sem_r)
    pltpu.async_copy(xt_hbm.at[pl.ds(f0, 1), :], idx_v, sem_i)
    sem_o = (sem_o0, sem_o1)

    def field_body(i, carry):
        f = lax.rem(d + i, NUM_FIELDS)
        pltpu.make_async_copy(xt_hbm.at[pl.ds(f, 1), :], idx_v, sem_i).wait()
        pltpu.make_async_copy(tt_hbm.at[f].at[d], row_v, sem_r).wait()

        for q in range(NQ):
            slot = q % 2
            out_desc = pltpu.make_async_copy(
                buf_v.at[pl.ds(slot, 1)],
                out_hbm.at[f].at[pl.ds(d, 1), pl.ds(q * QB, QB)],
                sem_o[slot],
            )
            if q >= 2:
                out_desc.wait()
            else:
                @pl.when(i > 0)
                def _():
                    out_desc.wait()

            @plsc.parallel_loop(0, QB // L, unroll=8)
            def gather_body(m):
                idx16 = idx_v[0, pl.ds(q * QB + m * L, L)]
                buf_v[slot, pl.ds(m * L, L)] = plsc.load_gather(row_v, [idx16])

            pltpu.async_copy(
                buf_v.at[pl.ds(slot, 1)],
                out_hbm.at[f].at[pl.ds(d, 1), pl.ds(q * QB, QB)],
                sem_o[slot],
            )

        @pl.when(i < NUM_FIELDS - 1)
        def _():
            fn = lax.rem(d + i + 1, NUM_FIELDS)
            pltpu.async_copy(tt_hbm.at[fn].at[d], row_v, sem_r)
            pltpu.async_copy(xt_hbm.at[pl.ds(fn, 1), :], idx_v, sem_i)

        return carry

    lax.fori_loop(0, NUM_FIELDS, field_body, 0)
    for slot in range(2):
        pltpu.make_async_copy(
            buf_v.at[pl.ds(slot, 1)],
            out_hbm.at[0].at[pl.ds(0, 1), pl.ds(0, QB)],
            sem_o[slot],
        ).wait()


def kernel(x, tables):
    xt = x.T
    tt = jnp.transpose(tables, (0, 2, 1))
    out_t = _emb_gather(xt, tt)
    return jnp.transpose(out_t, (2, 0, 1))


# speedup vs baseline: 23.1955x; 1.0010x over previous
import functools

import jax
import jax.numpy as jnp
from jax import lax
from jax.experimental import pallas as pl
from jax.experimental.pallas import tpu as pltpu
from jax.experimental.pallas import tpu_sc as plsc

NUM_FIELDS = 26
CARD1 = 100001
D = 32
BATCH = 16384
NC, NS, L = 2, 16, 16
NW = NC * NS
QB = 4096
NQ = BATCH // QB

_mesh = plsc.VectorSubcoreMesh(
    core_axis_name="c", subcore_axis_name="s", num_cores=NC, num_subcores=NS
)


@functools.partial(
    pl.kernel,
    out_type=jax.ShapeDtypeStruct((NUM_FIELDS, D, BATCH), jnp.float32),
    mesh=_mesh,
    scratch_types=[
        pltpu.VMEM((CARD1,), jnp.float32),
        pltpu.VMEM((1, BATCH), jnp.int32),
        pltpu.VMEM((2, QB), jnp.float32),
        pltpu.SemaphoreType.DMA,
        pltpu.SemaphoreType.DMA,
        pltpu.SemaphoreType.DMA,
        pltpu.SemaphoreType.DMA,
    ],
    compiler_params=pltpu.CompilerParams(needs_layout_passes=False),
)
def _emb_gather(xt_hbm, tt_hbm, out_hbm, row_v, idx_v, buf_v, sem_r, sem_i,
                sem_o0, sem_o1):
    d = lax.axis_index("s") * NC + lax.axis_index("c")
    f0 = lax.rem(d, NUM_FIELDS)
    pltpu.async_copy(tt_hbm.at[f0].at[d], row_v, sem_r)
    pltpu.async_copy(xt_hbm.at[pl.ds(f0, 1), :], idx_v, sem_i)
    sem_o = (sem_o0, sem_o1)

    def field_body(i, carry):
        f = lax.rem(d + i, NUM_FIELDS)
        pltpu.make_async_copy(xt_hbm.at[pl.ds(f, 1), :], idx_v, sem_i).wait()
        pltpu.make_async_copy(tt_hbm.at[f].at[d], row_v, sem_r).wait()

        for q in range(NQ):
            slot = q % 2
            out_desc = pltpu.make_async_copy(
                buf_v.at[pl.ds(slot, 1)],
                out_hbm.at[f].at[pl.ds(d, 1), pl.ds(q * QB, QB)],
                sem_o[slot],
            )
            if q >= 2:
                out_desc.wait()
            else:
                @pl.when(i > 0)
                def _():
                    out_desc.wait()

            @plsc.parallel_loop(0, QB // L, unroll=16)
            def gather_body(m):
                idx16 = idx_v[0, pl.ds(q * QB + m * L, L)]
                buf_v[slot, pl.ds(m * L, L)] = plsc.load_gather(row_v, [idx16])

            pltpu.async_copy(
                buf_v.at[pl.ds(slot, 1)],
                out_hbm.at[f].at[pl.ds(d, 1), pl.ds(q * QB, QB)],
                sem_o[slot],
            )

        @pl.when(i < NUM_FIELDS - 1)
        def _():
            fn = lax.rem(d + i + 1, NUM_FIELDS)
            pltpu.async_copy(tt_hbm.at[fn].at[d], row_v, sem_r)
            pltpu.async_copy(xt_hbm.at[pl.ds(fn, 1), :], idx_v, sem_i)

        return carry

    lax.fori_loop(0, NUM_FIELDS, field_body, 0)
    for slot in range(2):
        pltpu.make_async_copy(
            buf_v.at[pl.ds(slot, 1)],
            out_hbm.at[0].at[pl.ds(0, 1), pl.ds(0, QB)],
            sem_o[slot],
        ).wait()


def kernel(x, tables):
    xt = x.T
    tt = jnp.transpose(tables, (0, 2, 1))
    out_t = _emb_gather(xt, tt)
    return jnp.transpose(out_t, (2, 0, 1))


# speedup vs baseline: 25.7295x; 1.1092x over previous
import functools

import jax
import jax.numpy as jnp
from jax import lax
from jax.experimental import pallas as pl
from jax.experimental.pallas import tpu as pltpu
from jax.experimental.pallas import tpu_sc as plsc

NUM_FIELDS = 26
CARD1 = 100001
D = 32
BATCH = 16384
NC, NS, L = 2, 16, 16
NW = NC * NS
QB = 4096
NQ = BATCH // QB

_mesh = plsc.VectorSubcoreMesh(
    core_axis_name="c", subcore_axis_name="s", num_cores=NC, num_subcores=NS
)


@functools.partial(
    pl.kernel,
    out_type=jax.ShapeDtypeStruct((NUM_FIELDS, D, BATCH), jnp.float32),
    mesh=_mesh,
    scratch_types=[
        pltpu.VMEM((CARD1,), jnp.float32),
        pltpu.VMEM((1, BATCH), jnp.int32),
        pltpu.VMEM((2, QB), jnp.float32),
        pltpu.VMEM_SHARED((2, BATCH), jnp.int32),
        pltpu.SemaphoreType.DMA,
        pltpu.SemaphoreType.DMA,
        pltpu.SemaphoreType.DMA,
        pltpu.SemaphoreType.DMA,
        pltpu.SemaphoreType.DMA,
    ],
    compiler_params=pltpu.CompilerParams(needs_layout_passes=False),
)
def _emb_gather(xt_hbm, tt_hbm, out_hbm, row_v, idx_v, buf_v, idx_sh, sem_r,
                sem_i, sem_l, sem_o0, sem_o1):
    s = lax.axis_index("s")
    d = s * NC + lax.axis_index("c")

    @pl.when(s == 0)
    def _():
        pltpu.sync_copy(xt_hbm.at[pl.ds(0, 1), :], idx_sh.at[pl.ds(0, 1), :])
        pltpu.sync_copy(xt_hbm.at[pl.ds(1, 1), :], idx_sh.at[pl.ds(1, 1), :])
    plsc.subcore_barrier()

    pltpu.async_copy(tt_hbm.at[0].at[d], row_v, sem_r)
    pltpu.async_copy(idx_sh.at[pl.ds(0, 1), :], idx_v, sem_i)
    sem_o = (sem_o0, sem_o1)

    def field_body(i, carry):
        f = i
        slot = lax.rem(i, 2)
        pltpu.make_async_copy(
            idx_sh.at[pl.ds(slot, 1), :], idx_v, sem_i
        ).wait()
        plsc.subcore_barrier()

        @pl.when((s == 0) & (i < NUM_FIELDS - 2))
        def _():
            pltpu.async_copy(
                xt_hbm.at[pl.ds(i + 2, 1), :],
                idx_sh.at[pl.ds(slot, 1), :],
                sem_l,
            )

        pltpu.make_async_copy(tt_hbm.at[f].at[d], row_v, sem_r).wait()

        for q in range(NQ):
            slot = q % 2
            out_desc = pltpu.make_async_copy(
                buf_v.at[pl.ds(slot, 1)],
                out_hbm.at[f].at[pl.ds(d, 1), pl.ds(q * QB, QB)],
                sem_o[slot],
            )
            if q >= 2:
                out_desc.wait()
            else:
                @pl.when(i > 0)
                def _():
                    out_desc.wait()

            @plsc.parallel_loop(0, QB // L, unroll=16)
            def gather_body(m):
                idx16 = idx_v[0, pl.ds(q * QB + m * L, L)]
                buf_v[slot, pl.ds(m * L, L)] = plsc.load_gather(row_v, [idx16])

            pltpu.async_copy(
                buf_v.at[pl.ds(slot, 1)],
                out_hbm.at[f].at[pl.ds(d, 1), pl.ds(q * QB, QB)],
                sem_o[slot],
            )

        @pl.when((s == 0) & (i < NUM_FIELDS - 2))
        def _():
            pltpu.make_async_copy(
                xt_hbm.at[pl.ds(i + 2, 1), :],
                idx_sh.at[pl.ds(slot, 1), :],
                sem_l,
            ).wait()

        plsc.subcore_barrier()

        @pl.when(i < NUM_FIELDS - 1)
        def _():
            pltpu.async_copy(tt_hbm.at[i + 1].at[d], row_v, sem_r)
            pltpu.async_copy(
                idx_sh.at[pl.ds(lax.rem(i + 1, 2), 1), :], idx_v, sem_i
            )

        return carry

    lax.fori_loop(0, NUM_FIELDS, field_body, 0)
    for slot in range(2):
        pltpu.make_async_copy(
            buf_v.at[pl.ds(slot, 1)],
            out_hbm.at[0].at[pl.ds(0, 1), pl.ds(0, QB)],
            sem_o[slot],
        ).wait()


def kernel(x, tables):
    xt = x.T
    tt = jnp.transpose(tables, (0, 2, 1))
    out_t = _emb_gather(xt, tt)
    return jnp.transpose(out_t, (2, 0, 1))


# speedup vs baseline: 26.3848x; 1.0255x over previous
import functools

import jax
import jax.numpy as jnp
from jax import lax
from jax.experimental import pallas as pl
from jax.experimental.pallas import tpu as pltpu
from jax.experimental.pallas import tpu_sc as plsc

NUM_FIELDS = 26
CARD1 = 100001
D = 32
BATCH = 16384
NC, NS, L = 2, 16, 16
NW = NC * NS
QB = 4096
NQ = BATCH // QB

_mesh = plsc.VectorSubcoreMesh(
    core_axis_name="c", subcore_axis_name="s", num_cores=NC, num_subcores=NS
)


@functools.partial(
    pl.kernel,
    out_type=jax.ShapeDtypeStruct((NUM_FIELDS, D, BATCH), jnp.float32),
    mesh=_mesh,
    scratch_types=[
        pltpu.VMEM((CARD1,), jnp.float32),
        pltpu.VMEM((1, BATCH), jnp.int32),
        pltpu.VMEM((2, QB), jnp.float32),
        pltpu.VMEM_SHARED((2, BATCH), jnp.int32),
        pltpu.SemaphoreType.DMA,
        pltpu.SemaphoreType.DMA,
        pltpu.SemaphoreType.DMA,
        pltpu.SemaphoreType.DMA,
        pltpu.SemaphoreType.DMA,
    ],
    compiler_params=pltpu.CompilerParams(needs_layout_passes=False),
)
def _emb_gather(xt_hbm, tt_hbm, out_hbm, row_v, idx_v, buf_v, idx_sh, sem_r,
                sem_i, sem_l, sem_o0, sem_o1):
    s = lax.axis_index("s")
    d = s * NC + lax.axis_index("c")

    @pl.when(s == 0)
    def _():
        pltpu.sync_copy(xt_hbm.at[pl.ds(0, 1), :], idx_sh.at[pl.ds(0, 1), :])
        pltpu.sync_copy(xt_hbm.at[pl.ds(1, 1), :], idx_sh.at[pl.ds(1, 1), :])
    plsc.subcore_barrier()

    pltpu.async_copy(tt_hbm.at[0].at[d], row_v, sem_r)
    pltpu.async_copy(idx_sh.at[pl.ds(0, 1), :], idx_v, sem_i)
    sem_o = (sem_o0, sem_o1)

    def field_body(i, carry):
        f = i
        slot = lax.rem(i, 2)
        pltpu.make_async_copy(
            idx_sh.at[pl.ds(slot, 1), :], idx_v, sem_i
        ).wait()
        plsc.subcore_barrier()

        @pl.when((s == 0) & (i < NUM_FIELDS - 2))
        def _():
            pltpu.async_copy(
                xt_hbm.at[pl.ds(i + 2, 1), :],
                idx_sh.at[pl.ds(slot, 1), :],
                sem_l,
            )

        pltpu.make_async_copy(tt_hbm.at[f].at[d], row_v, sem_r).wait()

        for q in range(NQ):
            slot = q % 2
            out_desc = pltpu.make_async_copy(
                buf_v.at[pl.ds(slot, 1)],
                out_hbm.at[f].at[pl.ds(d, 1), pl.ds(q * QB, QB)],
                sem_o[slot],
            )
            if q >= 2:
                out_desc.wait()
            else:
                @pl.when(i > 0)
                def _():
                    out_desc.wait()

            @plsc.parallel_loop(0, QB // L, unroll=16)
            def gather_body(m):
                idx16 = idx_v[0, pl.ds(q * QB + m * L, L)]
                buf_v[slot, pl.ds(m * L, L)] = plsc.load_gather(row_v, [idx16])

            pltpu.async_copy(
                buf_v.at[pl.ds(slot, 1)],
                out_hbm.at[f].at[pl.ds(d, 1), pl.ds(q * QB, QB)],
                sem_o[slot],
            )

        @pl.when(i < NUM_FIELDS - 1)
        def _():
            pltpu.async_copy(tt_hbm.at[i + 1].at[d], row_v, sem_r)

        @pl.when((s == 0) & (i < NUM_FIELDS - 2))
        def _():
            pltpu.make_async_copy(
                xt_hbm.at[pl.ds(i + 2, 1), :],
                idx_sh.at[pl.ds(slot, 1), :],
                sem_l,
            ).wait()

        plsc.subcore_barrier()

        @pl.when(i < NUM_FIELDS - 1)
        def _():
            pltpu.async_copy(
                idx_sh.at[pl.ds(lax.rem(i + 1, 2), 1), :], idx_v, sem_i
            )

        return carry

    lax.fori_loop(0, NUM_FIELDS, field_body, 0)
    for slot in range(2):
        pltpu.make_async_copy(
            buf_v.at[pl.ds(slot, 1)],
            out_hbm.at[0].at[pl.ds(0, 1), pl.ds(0, QB)],
            sem_o[slot],
        ).wait()


def kernel(x, tables):
    xt = x.T
    tt = jnp.transpose(tables, (0, 2, 1))
    out_t = _emb_gather(xt, tt)
    return jnp.transpose(out_t, (2, 0, 1))
